# baseline (device time: 39618 ns/iter reference)
import jax
import jax.numpy as jnp
from jax import lax
from jax.experimental import pallas as pl
from jax.experimental.pallas import tpu as pltpu

M = 1024
D = 1024
HALF = M // 2
CHUNKS = (80, 80, 80, 80, 80, 64, 32, 16)
assert sum(CHUNKS) == HALF
K = len(CHUNKS)
EPS = 1e-6


def kernel(partial, resid, gamma):
    x2d = partial.reshape(M, D)
    g2d = gamma.reshape(1, D)

    def body(
        x_hbm, r_hbm, g_hbm, out_hbm,
        xv, rv, gv, recv, outv,
        lsem, wsem, s1, r1, s2, r2,
    ):
        my_x = lax.axis_index("x")
        my_y = lax.axis_index("y")
        h = (my_x + my_y) % 2
        off_mine = h * HALF
        off_other = (1 - h) * HALF

        xnbr = (1 - my_x, my_y)
        ynbr = (my_x, 1 - my_y)

        barrier = pltpu.get_barrier_semaphore()
        for nbr in (xnbr, ynbr):
            pl.semaphore_signal(
                barrier, inc=1, device_id=nbr,
                device_id_type=pl.DeviceIdType.MESH,
            )
        pl.semaphore_wait(barrier, 2)

        p1 = []
        o = 0
        for c, ch in enumerate(CHUNKS):
            rdma = pltpu.make_async_remote_copy(
                src_ref=x_hbm.at[pl.ds(off_other + o, ch), :],
                dst_ref=recv.at[pl.ds(o, ch), :],
                send_sem=s1.at[c],
                recv_sem=r1.at[c],
                device_id=xnbr,
                device_id_type=pl.DeviceIdType.MESH,
            )
            rdma.start()
            p1.append(rdma)
            o += ch

        cp_x = pltpu.make_async_copy(
            x_hbm.at[pl.ds(off_mine, HALF), :], xv, lsem.at[0])
        cp_r = pltpu.make_async_copy(
            r_hbm.at[pl.ds(off_mine, HALF), :], rv, lsem.at[1])
        cp_g = pltpu.make_async_copy(g_hbm, gv, lsem.at[2])
        cp_x.start()
        cp_r.start()
        cp_g.start()
        cp_x.wait()
        cp_r.wait()
        cp_g.wait()

        p2 = []
        wb = []
        o = 0
        for c, ch in enumerate(CHUNKS):
            p1[c].wait_recv()
            sl = pl.ds(o, ch)
            ysum = xv[sl, :] + recv[sl, :] + rv[sl, :]
            ms = jnp.mean(ysum * ysum, axis=-1, keepdims=True)
            outv[sl, :] = ysum * lax.rsqrt(ms + EPS) * gv[...]
            row = off_mine + o
            rdma2 = pltpu.make_async_remote_copy(
                src_ref=outv.at[sl, :],
                dst_ref=out_hbm.at[pl.ds(row, ch), :],
                send_sem=s2.at[c],
                recv_sem=r2.at[c],
                device_id=ynbr,
                device_id_type=pl.DeviceIdType.MESH,
            )
            rdma2.start()
            p2.append(rdma2)
            w = pltpu.make_async_copy(
                outv.at[sl, :], out_hbm.at[pl.ds(row, ch), :], wsem.at[c])
            w.start()
            wb.append(w)
            o += ch

        for c in range(K):
            p2[c].wait_recv()
            p1[c].wait_send()
            p2[c].wait_send()
            wb[c].wait()

    return pl.pallas_call(
        body,
        out_shape=jax.ShapeDtypeStruct((M, D), jnp.float32),
        in_specs=[pl.BlockSpec(memory_space=pl.ANY)] * 3,
        out_specs=pl.BlockSpec(memory_space=pl.ANY),
        scratch_shapes=[
            pltpu.VMEM((HALF, D), jnp.float32),
            pltpu.VMEM((HALF, D), jnp.float32),
            pltpu.VMEM((1, D), jnp.float32),
            pltpu.VMEM((HALF, D), jnp.float32),
            pltpu.VMEM((HALF, D), jnp.float32),
            pltpu.SemaphoreType.DMA((3,)),
            pltpu.SemaphoreType.DMA((K,)),
            pltpu.SemaphoreType.DMA((K,)),
            pltpu.SemaphoreType.DMA((K,)),
            pltpu.SemaphoreType.DMA((K,)),
            pltpu.SemaphoreType.DMA((K,)),
        ],
        compiler_params=pltpu.CompilerParams(collective_id=0),
    )(x2d, resid, g2d)


# device time: 37640 ns/iter; 1.0526x vs baseline; 1.0526x over previous
import jax
import jax.numpy as jnp
from jax import lax
from jax.experimental import pallas as pl
from jax.experimental.pallas import tpu as pltpu

M = 1024
D = 1024
HALF = M // 2
CHUNKS = (32,) * 16
assert sum(CHUNKS) == HALF
K = len(CHUNKS)
EPS = 1e-6


def kernel(partial, resid, gamma):
    def body(x_ref, r_ref, g_ref, out_ref, recv, sumv, s1, r1, s2, r2):
        my_x = lax.axis_index("x")
        my_y = lax.axis_index("y")
        h = (my_x + my_y) % 2
        off_mine = h * HALF
        off_other = (1 - h) * HALF

        xnbr = (1 - my_x, my_y)
        ynbr = (my_x, 1 - my_y)

        barrier = pltpu.get_barrier_semaphore()
        pl.semaphore_signal(
            barrier, inc=2, device_id=xnbr,
            device_id_type=pl.DeviceIdType.MESH,
        )
        pl.semaphore_signal(
            barrier, inc=1, device_id=ynbr,
            device_id_type=pl.DeviceIdType.MESH,
        )
        pl.semaphore_wait(barrier, 2)

        p1 = []
        o = 0
        for c, ch in enumerate(CHUNKS):
            rdma = pltpu.make_async_remote_copy(
                src_ref=x_ref.at[0, pl.ds(off_other + o, ch), :],
                dst_ref=recv.at[pl.ds(o, ch), :],
                send_sem=s1.at[c],
                recv_sem=r1.at[c],
                device_id=xnbr,
                device_id_type=pl.DeviceIdType.MESH,
            )
            rdma.start()
            p1.append(rdma)
            o += ch

        p2 = []
        o = 0
        for c, ch in enumerate(CHUNKS):
            p1[c].wait_recv()
            sl = pl.ds(off_mine + o, ch)
            slo = pl.ds(o, ch)
            recv[slo, :] = x_ref[0, sl, :] + recv[slo, :] + r_ref[sl, :]
            if c == 0:
                pl.semaphore_wait(barrier, 1)
            rdma2 = pltpu.make_async_remote_copy(
                src_ref=recv.at[slo, :],
                dst_ref=sumv.at[slo, :],
                send_sem=s2.at[c],
                recv_sem=r2.at[c],
                device_id=ynbr,
                device_id_type=pl.DeviceIdType.MESH,
            )
            rdma2.start()
            p2.append(rdma2)
            o += ch

        g = g_ref[...].reshape(1, D)

        o = 0
        for c, ch in enumerate(CHUNKS):
            ysum = recv[pl.ds(o, ch), :]
            ms = jnp.mean(ysum * ysum, axis=-1, keepdims=True)
            out_ref[pl.ds(off_mine + o, ch), :] = (
                ysum * lax.rsqrt(ms + EPS) * g
            )
            o += ch

        o = 0
        for c, ch in enumerate(CHUNKS):
            p2[c].wait_recv()
            ysum = sumv[pl.ds(o, ch), :]
            ms = jnp.mean(ysum * ysum, axis=-1, keepdims=True)
            out_ref[pl.ds(off_other + o, ch), :] = (
                ysum * lax.rsqrt(ms + EPS) * g
            )
            o += ch

        for c in range(K):
            p1[c].wait_send()
            p2[c].wait_send()

    return pl.pallas_call(
        body,
        out_shape=jax.ShapeDtypeStruct((M, D), jnp.float32),
        in_specs=[pl.BlockSpec(memory_space=pltpu.VMEM)] * 3,
        out_specs=pl.BlockSpec(memory_space=pltpu.VMEM),
        scratch_shapes=[
            pltpu.VMEM((HALF, D), jnp.float32),
            pltpu.VMEM((HALF, D), jnp.float32),
            pltpu.SemaphoreType.DMA((K,)),
            pltpu.SemaphoreType.DMA((K,)),
            pltpu.SemaphoreType.DMA((K,)),
            pltpu.SemaphoreType.DMA((K,)),
        ],
        compiler_params=pltpu.CompilerParams(collective_id=0),
    )(partial, resid, gamma)
